# baseline (speedup 1.0000x reference)
import jax
import jax.numpy as jnp
from jax.experimental import pallas as pl
from jax.experimental.pallas import tpu as pltpu
from jax.experimental.pallas import tpu_sc as plsc

EMB = 256
KCODES = 8192
NTOK = 8192
TBLK = 256
NBLK = NTOK // TBLK
TOK_PER_B = 1024
BLK_PER_B = TOK_PER_B // TBLK
GATHER_W = 128


def _vq_body(z_ref, z2_ref, cb_ref, c2_ref, idx_ref, hist_ref, cnt_ref):
    zb = z_ref[...]
    cb = cb_ref[...]
    mm = jax.lax.dot_general(
        zb, cb, (((1,), (1,)), ((), ())),
        precision=jax.lax.Precision.DEFAULT,
        preferred_element_type=jnp.float32)
    dist = (z2_ref[...] + c2_ref[...]) - 2.0 * mm
    m = jnp.min(dist, axis=1, keepdims=True)
    lane = jax.lax.broadcasted_iota(jnp.int32, dist.shape, 1)
    cand = jnp.where(dist == m, lane, jnp.int32(KCODES))
    idx = jnp.min(cand, axis=1)
    idx_ref[0, 0, :] = idx
    e = jnp.exp(m - dist)
    s = jnp.sum(e, axis=1, keepdims=True)
    prob = e * (1.0 / s)
    hist_ref[0, 0, :] = jnp.sum(prob, axis=0)
    oh_tk = (cand == idx[:, None])
    cnt_ref[0, 0, :] = jnp.sum(oh_tk.astype(jnp.float32), axis=0)


KB = 1024


def _oh_body(idx_ref, oh_ref):
    k0 = pl.program_id(1) * KB
    kio = jax.lax.broadcasted_iota(jnp.int32, (KB, TOK_PER_B), 0) + k0
    oh = (kio == idx_ref[0, 0, :][None, :]).astype(jnp.int32)
    oh_ref[0] = oh.reshape(KB, 32, 32)


def _oh_call(idx_rows):
    return pl.pallas_call(
        _oh_body,
        grid=(8, KCODES // KB),
        in_specs=[pl.BlockSpec((1, 1, TOK_PER_B), lambda b, k: (b, 0, 0))],
        out_specs=pl.BlockSpec((1, KB, 32, 32), lambda b, k: (b, k, 0, 0)),
        out_shape=jax.ShapeDtypeStruct((8, KCODES, 32, 32), jnp.int32),
        compiler_params=pltpu.CompilerParams(
            dimension_semantics=("parallel", "parallel")),
    )(idx_rows)


def _fin_body(zq_ref, z_ref, zqo_ref, sse_ref):
    zq = zq_ref[...]
    zqt = zq.T
    zb = z_ref[0]
    d = zqt - zb
    zqo_ref[0] = zb + d
    sse_ref[...] = jnp.sum(d * d).reshape(1, 1, 1)


def _sc_gather(codebook, idx_flat):
    mesh = plsc.VectorSubcoreMesh(core_axis_name="core",
                                  subcore_axis_name="subcore")

    @pl.kernel(out_type=jax.ShapeDtypeStruct((NTOK, EMB), jnp.float32),
               mesh=mesh)
    def kern(cb_hbm, i_hbm, o_hbm):
        def body(i_vmem, o_vmem):
            pltpu.sync_copy(cb_hbm.at[i_vmem.at[0]], o_vmem)

        pltpu.emit_pipeline(
            body,
            grid=(NTOK // GATHER_W,),
            in_specs=[pl.BlockSpec((1, GATHER_W), index_map=lambda i: (0, i))],
            out_specs=[pl.BlockSpec((GATHER_W, EMB),
                                    index_map=lambda i: (i, 0))],
            core_axis_name=("core", "subcore"),
            dimension_semantics=(pltpu.PARALLEL,),
        )(i_hbm, o_hbm)

    return kern(codebook, idx_flat)


def _main_call(z_flat, z2, codebook, c2):
    return pl.pallas_call(
        _vq_body,
        grid=(NBLK,),
        in_specs=[
            pl.BlockSpec((TBLK, EMB), lambda i: (i, 0)),
            pl.BlockSpec((TBLK, 1), lambda i: (i, 0)),
            pl.BlockSpec((KCODES, EMB), lambda i: (0, 0)),
            pl.BlockSpec((1, KCODES), lambda i: (0, 0)),
        ],
        out_specs=[
            pl.BlockSpec((1, 1, TBLK), lambda i: (i, 0, 0)),
            pl.BlockSpec((1, 1, KCODES), lambda i: (i, 0, 0)),
            pl.BlockSpec((1, 1, KCODES), lambda i: (i, 0, 0)),
        ],
        out_shape=[
            jax.ShapeDtypeStruct((NBLK, 1, TBLK), jnp.int32),
            jax.ShapeDtypeStruct((NBLK, 1, KCODES), jnp.float32),
            jax.ShapeDtypeStruct((NBLK, 1, KCODES), jnp.float32),
        ],
        compiler_params=pltpu.CompilerParams(
            dimension_semantics=("parallel",)),
    )(z_flat, z2, codebook, c2)


def _fin_call(zq_flat, zr):
    return pl.pallas_call(
        _fin_body,
        grid=(NBLK,),
        in_specs=[
            pl.BlockSpec((TBLK, EMB), lambda i: (i, 0)),
            pl.BlockSpec((1, EMB, TBLK),
                         lambda i: (i // BLK_PER_B, 0, i % BLK_PER_B)),
        ],
        out_specs=[
            pl.BlockSpec((1, EMB, TBLK),
                         lambda i: (i // BLK_PER_B, 0, i % BLK_PER_B)),
            pl.BlockSpec((1, 1, 1), lambda i: (i, 0, 0)),
        ],
        out_shape=[
            jax.ShapeDtypeStruct((8, EMB, TOK_PER_B), jnp.float32),
            jax.ShapeDtypeStruct((NBLK, 1, 1), jnp.float32),
        ],
        compiler_params=pltpu.CompilerParams(
            dimension_semantics=("parallel",)),
    )(zq_flat, zr)


def kernel(z, codebook):
    B = z.shape[0]
    spatial = z.shape[2:]
    z_flat = jnp.moveaxis(z, 1, -1).reshape(-1, EMB)
    z2 = jnp.sum(z_flat ** 2, axis=1, keepdims=True)
    c2 = jnp.sum(codebook ** 2, axis=1)[None, :]

    idxp, histp, cntp = _main_call(z_flat, z2, codebook, c2)
    oh = _oh_call(idxp.reshape(B, 1, TOK_PER_B))

    idx_flat = idxp.reshape(1, NTOK)
    zq_flat = _sc_gather(codebook, idx_flat)

    zr = z.reshape(B, EMB, TOK_PER_B)
    zqo, ssep = _fin_call(zq_flat, zr)

    mse = jnp.sum(ssep) / (B * EMB * spatial[0] * spatial[1])
    loss = mse + 0.25 * mse
    counts = jnp.sum(cntp[:, 0, :], axis=0)
    avg_probs = counts / NTOK
    perplexity = jnp.exp(-jnp.sum(avg_probs * jnp.log(avg_probs + 1e-10)))
    hist = jnp.sum(histp[:, 0, :].reshape(B, BLK_PER_B, KCODES), axis=1)

    z_quantized_ste = zqo.reshape(B, EMB, *spatial)
    idx_out = idxp.reshape(B, *spatial)
    return (loss, z_quantized_ste, perplexity, oh, idx_out, hist)


# speedup vs baseline: 2.4851x; 2.4851x over previous
import jax
import jax.numpy as jnp
from jax.experimental import pallas as pl
from jax.experimental.pallas import tpu as pltpu
from jax.experimental.pallas import tpu_sc as plsc

EMB = 256
KCODES = 8192
NTOK = 8192
TBLK = 256
NBLK = NTOK // TBLK
TOK_PER_B = 1024
BLK_PER_B = TOK_PER_B // TBLK
GATHER_W = 128


def _vq_body(z_ref, z2_ref, cb_ref, c2_ref, oh_ref, idx_ref, hist_ref, cnt_ref):
    zb = z_ref[...]
    cb = cb_ref[...]
    mm = jax.lax.dot_general(
        zb, cb, (((1,), (1,)), ((), ())),
        precision=jax.lax.Precision.DEFAULT,
        preferred_element_type=jnp.float32)
    dist = (z2_ref[...] + c2_ref[...]) - 2.0 * mm
    m = jnp.min(dist, axis=1, keepdims=True)
    lane = jax.lax.broadcasted_iota(jnp.int32, dist.shape, 1)
    cand = jnp.where(dist == m, lane, jnp.int32(KCODES))
    idx = jnp.min(cand, axis=1)
    idx_ref[0, 0, :] = idx
    e = jnp.exp(m - dist)
    s = jnp.sum(e, axis=1, keepdims=True)
    prob = e * (1.0 / s)
    hist_ref[0, 0, :] = jnp.sum(prob, axis=0)
    oh_tk = (cand == idx[:, None])
    cnt_ref[0, 0, :] = jnp.sum(oh_tk.astype(jnp.float32), axis=0)
    kio = jax.lax.broadcasted_iota(jnp.int32, (KCODES, TBLK), 0)
    oh_ref[0] = (kio == idx[None, :]).astype(jnp.int32)


def _fin_body(zq_ref, z_ref, zqo_ref, sse_ref):
    zq = zq_ref[...]
    zqt = zq.T
    zb = z_ref[0]
    d = zqt - zb
    zqo_ref[0] = zb + d
    sse_ref[...] = jnp.sum(d * d).reshape(1, 1, 1)


def _sc_gather(codebook, idx_flat):
    mesh = plsc.VectorSubcoreMesh(core_axis_name="core",
                                  subcore_axis_name="subcore")

    @pl.kernel(out_type=jax.ShapeDtypeStruct((NTOK, EMB), jnp.float32),
               mesh=mesh)
    def kern(cb_hbm, i_hbm, o_hbm):
        def body(i_vmem, o_vmem):
            pltpu.sync_copy(cb_hbm.at[i_vmem.at[0]], o_vmem)

        pltpu.emit_pipeline(
            body,
            grid=(NTOK // GATHER_W,),
            in_specs=[pl.BlockSpec((1, GATHER_W), index_map=lambda i: (0, i))],
            out_specs=[pl.BlockSpec((GATHER_W, EMB),
                                    index_map=lambda i: (i, 0))],
            core_axis_name=("core", "subcore"),
            dimension_semantics=(pltpu.PARALLEL,),
        )(i_hbm, o_hbm)

    return kern(codebook, idx_flat)


def _main_call(z_flat, z2, codebook, c2):
    return pl.pallas_call(
        _vq_body,
        grid=(NBLK,),
        in_specs=[
            pl.BlockSpec((TBLK, EMB), lambda i: (i, 0)),
            pl.BlockSpec((TBLK, 1), lambda i: (i, 0)),
            pl.BlockSpec((KCODES, EMB), lambda i: (0, 0)),
            pl.BlockSpec((1, KCODES), lambda i: (0, 0)),
        ],
        out_specs=[
            pl.BlockSpec((1, KCODES, TBLK),
                         lambda i: (i // BLK_PER_B, 0, i % BLK_PER_B)),
            pl.BlockSpec((1, 1, TBLK), lambda i: (i, 0, 0)),
            pl.BlockSpec((1, 1, KCODES), lambda i: (i, 0, 0)),
            pl.BlockSpec((1, 1, KCODES), lambda i: (i, 0, 0)),
        ],
        out_shape=[
            jax.ShapeDtypeStruct((8, KCODES, TOK_PER_B), jnp.int32),
            jax.ShapeDtypeStruct((NBLK, 1, TBLK), jnp.int32),
            jax.ShapeDtypeStruct((NBLK, 1, KCODES), jnp.float32),
            jax.ShapeDtypeStruct((NBLK, 1, KCODES), jnp.float32),
        ],
        compiler_params=pltpu.CompilerParams(
            dimension_semantics=("parallel",)),
    )(z_flat, z2, codebook, c2)


def _fin_call(zq_flat, zr):
    return pl.pallas_call(
        _fin_body,
        grid=(NBLK,),
        in_specs=[
            pl.BlockSpec((TBLK, EMB), lambda i: (i, 0)),
            pl.BlockSpec((1, EMB, TBLK),
                         lambda i: (i // BLK_PER_B, 0, i % BLK_PER_B)),
        ],
        out_specs=[
            pl.BlockSpec((1, EMB, TBLK),
                         lambda i: (i // BLK_PER_B, 0, i % BLK_PER_B)),
            pl.BlockSpec((1, 1, 1), lambda i: (i, 0, 0)),
        ],
        out_shape=[
            jax.ShapeDtypeStruct((8, EMB, TOK_PER_B), jnp.float32),
            jax.ShapeDtypeStruct((NBLK, 1, 1), jnp.float32),
        ],
        compiler_params=pltpu.CompilerParams(
            dimension_semantics=("parallel",)),
    )(zq_flat, zr)


def kernel(z, codebook):
    B = z.shape[0]
    spatial = z.shape[2:]
    z_flat = jnp.moveaxis(z, 1, -1).reshape(-1, EMB)
    z2 = jnp.sum(z_flat ** 2, axis=1, keepdims=True)
    c2 = jnp.sum(codebook ** 2, axis=1)[None, :]

    oh, idxp, histp, cntp = _main_call(z_flat, z2, codebook, c2)

    idx_flat = idxp.reshape(1, NTOK)
    zq_flat = _sc_gather(codebook, idx_flat)

    zr = z.reshape(B, EMB, TOK_PER_B)
    zqo, ssep = _fin_call(zq_flat, zr)

    mse = jnp.sum(ssep) / (B * EMB * spatial[0] * spatial[1])
    loss = mse + 0.25 * mse
    counts = jnp.sum(cntp[:, 0, :], axis=0)
    avg_probs = counts / NTOK
    perplexity = jnp.exp(-jnp.sum(avg_probs * jnp.log(avg_probs + 1e-10)))
    hist = jnp.sum(histp[:, 0, :].reshape(B, BLK_PER_B, KCODES), axis=1)

    z_quantized_ste = zqo.reshape(B, EMB, *spatial)
    onehot_out = oh.reshape(B, KCODES, *spatial)
    idx_out = idxp.reshape(B, *spatial)
    return (loss, z_quantized_ste, perplexity, onehot_out, idx_out, hist)
